# baseline (speedup 1.0000x reference)
import functools

import jax
import jax.numpy as jnp
from jax import lax
from jax.experimental import pallas as pl
from jax.experimental.pallas import tpu as pltpu
from jax.experimental.pallas import tpu_sc as plsc

_T = 2048
_D = 1024
_DH = 512
_DS = 2048
_E = 8
_NULL_COPIES = 8
_RHO = 0.5

_BLK = 256
_NB = 24
_DR = _NB * _BLK
_DR_TOT = _DR + _BLK
_TRASH = _DR
_NW = 32
_TPW = _T // _NW
_CH = 32


def _silu(v):
    return v * jax.nn.sigmoid(v)


def _gating_body(x_ref, gw_ref, bias_ref, null_ref, rs1_ref, rs2_ref,
                 rc1_ref, rc2_ref, wc1_ref, wc2_ref, be_ref, nb_ref, aux_ref):
    x = x_ref[...]
    logits = jnp.dot(x, gw_ref[...], preferred_element_type=jnp.float32)
    logits = logits + bias_ref[...]
    n = null_ref[0, 0]
    T, E = logits.shape

    iota = lax.broadcasted_iota(jnp.int32, (T, E), 1)
    v1 = jnp.max(logits, axis=1, keepdims=True)
    e1 = jnp.min(jnp.where(logits == v1, iota, E), axis=1, keepdims=True)
    l2 = jnp.where(iota == e1, -jnp.inf, logits)
    v2 = jnp.max(l2, axis=1, keepdims=True)
    e2 = jnp.min(jnp.where(l2 == v2, iota, E), axis=1, keepdims=True)

    t1_real = v1 >= n
    t2_real = jnp.logical_and(t1_real, v2 >= n)

    w1_both = 1.0 / (1.0 + jnp.exp(v2 - v1))
    w1 = jnp.where(t1_real, jnp.where(t2_real, w1_both, 1.0), 0.0)
    w2 = jnp.where(t2_real, 1.0 - w1_both, 0.0)
    wc1_ref[...] = jnp.broadcast_to(w1, (T, 128))
    wc2_ref[...] = jnp.broadcast_to(w2, (T, 128))

    f32 = jnp.float32
    bf = jnp.bfloat16
    oh1 = jnp.where(jnp.logical_and(iota == e1, t1_real), 1.0, 0.0)
    oh2 = jnp.where(jnp.logical_and(iota == e2, t2_real), 1.0, 0.0)
    hcnt = oh1 + oh2

    r_io = lax.broadcasted_iota(jnp.int32, (T, T), 0)
    c_io = lax.broadcasted_iota(jnp.int32, (T, T), 1)
    lst = jnp.where(r_io > c_io, 1.0, 0.0).astype(bf)
    cpre = jnp.dot(lst, hcnt.astype(bf), preferred_element_type=f32)

    tot = jnp.sum(hcnt, axis=0, keepdims=True)
    padded = jnp.floor((tot + (_BLK - 1)) * (1.0 / _BLK)) * _BLK
    er = lax.broadcasted_iota(jnp.int32, (E, E), 0)
    ec = lax.broadcasted_iota(jnp.int32, (E, E), 1)
    su = jnp.where(er < ec, 1.0, 0.0).astype(bf)
    base = jnp.dot(padded.astype(bf), su, preferred_element_type=f32)

    pos1 = jnp.sum(oh1 * (cpre + base), axis=1, keepdims=True)
    pos2 = jnp.sum(oh2 * (cpre + base), axis=1, keepdims=True)
    p1i = pos1.astype(jnp.int32)
    p2i = pos2.astype(jnp.int32)
    rs1_ref[...] = jnp.where(t1_real, p1i, _TRASH)
    rs2_ref[...] = jnp.where(t2_real, p2i, _TRASH)
    rc1_ref[...] = jnp.where(t1_real, p1i, _DR)
    rc2_ref[...] = jnp.where(t2_real, p2i, _DR)

    ends = base + padded
    jv = (lax.broadcasted_iota(jnp.int32, (_NB + 1, E), 0) * _BLK).astype(f32)
    beq = jnp.sum(jnp.where(jnp.broadcast_to(ends, (_NB + 1, E)) <= jv, 1, 0),
                  axis=1, keepdims=True)
    ev = lax.broadcasted_iota(jnp.int32, (1, E), 1)
    last_e = jnp.max(jnp.where(padded > 0, ev, 0))
    be_ref[...] = jnp.minimum(beq, last_e).reshape(1, _NB + 1)
    nb_ref[0, 0] = (jnp.sum(padded) * (1.0 / _BLK)).astype(jnp.int32)

    p = jnp.exp(logits - v1)
    probs_real = p / jnp.sum(p, axis=1, keepdims=True)
    P_real = jnp.sum(probs_real, axis=0) / T

    counts = jnp.sum(hcnt, axis=0)
    total_real = jnp.maximum(jnp.sum(counts), 1e-6)
    f_real = counts / total_real
    L_bal = E * jnp.sum(f_real * P_real)

    null_slots = jnp.sum(jnp.where(t1_real, 0.0, 1.0) + jnp.where(t2_real, 0.0, 1.0))
    null_rate = null_slots / (T * 2)
    L_null = (null_rate - _RHO) ** 2

    m = jnp.maximum(v1, n)
    s_all = (jnp.sum(jnp.exp(logits - m), axis=1, keepdims=True)
             + _NULL_COPIES * jnp.exp(n - m))
    lse = m + jnp.log(s_all)
    L_z = jnp.sum(lse * lse) / T

    aux_ref[0, 0] = 0.02 * L_bal + 0.001 * L_z + 0.01 * L_null


def _gating(x2d, gate_w, logit_bias, null_logit):
    i32 = jnp.int32
    f32 = jnp.float32
    return pl.pallas_call(
        _gating_body,
        out_shape=(
            jax.ShapeDtypeStruct((_T, 1), i32),
            jax.ShapeDtypeStruct((_T, 1), i32),
            jax.ShapeDtypeStruct((_T, 1), i32),
            jax.ShapeDtypeStruct((_T, 1), i32),
            jax.ShapeDtypeStruct((_T, 128), f32),
            jax.ShapeDtypeStruct((_T, 128), f32),
            jax.ShapeDtypeStruct((1, _NB + 1), i32),
            jax.ShapeDtypeStruct((1, 1), i32),
            jax.ShapeDtypeStruct((1, 1), f32),
        ),
        in_specs=[pl.BlockSpec(memory_space=pltpu.VMEM)] * 4,
        out_specs=(
            pl.BlockSpec(memory_space=pltpu.VMEM),
            pl.BlockSpec(memory_space=pltpu.VMEM),
            pl.BlockSpec(memory_space=pltpu.VMEM),
            pl.BlockSpec(memory_space=pltpu.VMEM),
            pl.BlockSpec(memory_space=pltpu.VMEM),
            pl.BlockSpec(memory_space=pltpu.VMEM),
            pl.BlockSpec(memory_space=pltpu.VMEM),
            pl.BlockSpec(memory_space=pltpu.SMEM),
            pl.BlockSpec(memory_space=pltpu.SMEM),
        ),
    )(x2d, gate_w, logit_bias.reshape(1, _E), null_logit.reshape(1, 1))


def _shared_body(x_ref, gw_ref, uw_ref, dw_ref, out_ref):
    x = x_ref[...].astype(jnp.bfloat16)
    g = jnp.dot(x, gw_ref[...].astype(jnp.bfloat16), preferred_element_type=jnp.float32)
    u = jnp.dot(x, uw_ref[...].astype(jnp.bfloat16), preferred_element_type=jnp.float32)
    h = (_silu(g) * u).astype(jnp.bfloat16)
    out_ref[...] = jnp.dot(h, dw_ref[...].astype(jnp.bfloat16),
                           preferred_element_type=jnp.float32)


def _shared_ffn(x2d, gw, uw, dw, tb=512):
    grid = (_T // tb,)
    return pl.pallas_call(
        _shared_body,
        grid=grid,
        out_shape=jax.ShapeDtypeStruct((_T, _D), jnp.float32),
        in_specs=[
            pl.BlockSpec((tb, _D), lambda t: (t, 0)),
            pl.BlockSpec((_D, _DS), lambda t: (0, 0)),
            pl.BlockSpec((_D, _DS), lambda t: (0, 0)),
            pl.BlockSpec((_DS, _D), lambda t: (0, 0)),
        ],
        out_specs=pl.BlockSpec((tb, _D), lambda t: (t, 0)),
    )(x2d, gw, uw, dw)


def _sc_wid():
    return lax.axis_index("s") * 2 + lax.axis_index("c")


def _scatter_body(x_hbm, w1_hbm, w2_hbm, rs1_hbm, rs2_hbm, xd_hbm, wd_hbm,
                  rows_v, wr1_v, wr2_v, idx1_v, idx2_v, sem):
    wid = _sc_wid()
    base = wid * _TPW
    pltpu.sync_copy(rs1_hbm.at[wid], idx1_v)
    pltpu.sync_copy(rs2_hbm.at[wid], idx2_v)
    for ch in range(_TPW // _CH):
        sl = pl.ds(base + ch * _CH, _CH)
        pltpu.sync_copy(x_hbm.at[sl, :], rows_v)
        pltpu.sync_copy(w1_hbm.at[sl, :], wr1_v)
        pltpu.sync_copy(w2_hbm.at[sl, :], wr2_v)
        cp1 = pltpu.async_copy(rows_v, xd_hbm.at[idx1_v.at[ch]], sem)
        cp2 = pltpu.async_copy(rows_v, xd_hbm.at[idx2_v.at[ch]], sem)
        cp3 = pltpu.async_copy(wr1_v, wd_hbm.at[idx1_v.at[ch]], sem)
        cp4 = pltpu.async_copy(wr2_v, wd_hbm.at[idx2_v.at[ch]], sem)
        cp1.wait()
        cp2.wait()
        cp3.wait()
        cp4.wait()


def _scatter(x2d, w1row, w2row, rs1, rs2):
    mesh = plsc.VectorSubcoreMesh(core_axis_name="c", subcore_axis_name="s")
    return pl.kernel(
        _scatter_body,
        out_type=(
            jax.ShapeDtypeStruct((_DR_TOT, _D), jnp.float32),
            jax.ShapeDtypeStruct((_DR_TOT, 128), jnp.float32),
        ),
        mesh=mesh,
        scratch_types=[
            pltpu.VMEM((_CH, _D), jnp.float32),
            pltpu.VMEM((_CH, 128), jnp.float32),
            pltpu.VMEM((_CH, 128), jnp.float32),
            pltpu.VMEM((_TPW // _CH, _CH), jnp.int32),
            pltpu.VMEM((_TPW // _CH, _CH), jnp.int32),
            pltpu.SemaphoreType.DMA,
        ],
    )(x2d, w1row, w2row, rs1, rs2)


def _experts_body(be_ref, nb_ref, xd_ref, wrow_ref, wg_ref, wu_ref, wd_ref,
                  out_ref):
    b = pl.program_id(0)

    @pl.when(b < nb_ref[0])
    def _():
        xb = xd_ref[...].astype(jnp.bfloat16)
        g = jnp.dot(xb, wg_ref[0].astype(jnp.bfloat16),
                    preferred_element_type=jnp.float32)
        u = jnp.dot(xb, wu_ref[0].astype(jnp.bfloat16),
                    preferred_element_type=jnp.float32)
        w = wrow_ref[:, 0:1]
        h = (_silu(g) * u * w).astype(jnp.bfloat16)
        out_ref[...] = jnp.dot(h, wd_ref[0].astype(jnp.bfloat16),
                               preferred_element_type=jnp.float32)

    @pl.when(b == _NB)
    def _zero_tail():
        out_ref[...] = jnp.zeros((_BLK, _D), jnp.float32)


def _experts(be, nb, xd, wd_rows, W_gate, W_up, W_down):
    grid_spec = pltpu.PrefetchScalarGridSpec(
        num_scalar_prefetch=2,
        grid=(_NB + 1,),
        in_specs=[
            pl.BlockSpec((_BLK, _D), lambda b, be, nb: (b, 0)),
            pl.BlockSpec((_BLK, 128), lambda b, be, nb: (b, 0)),
            pl.BlockSpec((1, _D, _DH), lambda b, be, nb: (be[b], 0, 0)),
            pl.BlockSpec((1, _D, _DH), lambda b, be, nb: (be[b], 0, 0)),
            pl.BlockSpec((1, _DH, _D), lambda b, be, nb: (be[b], 0, 0)),
        ],
        out_specs=pl.BlockSpec((_BLK, _D), lambda b, be, nb: (b, 0)),
    )
    return pl.pallas_call(
        _experts_body,
        grid_spec=grid_spec,
        out_shape=jax.ShapeDtypeStruct((_DR_TOT, _D), jnp.float32),
    )(be, nb, xd, wd_rows, W_gate, W_up, W_down)


def _combine_body(xo_hbm, sh_hbm, rc1_hbm, rc2_hbm, y_hbm,
                  acc_v, g1_v, g2_v, idx1_v, idx2_v, sem):
    wid = _sc_wid()
    base = wid * _TPW
    for ch in range(_TPW // _CH):
        sl = pl.ds(base + ch * _CH, _CH)
        pltpu.sync_copy(sh_hbm.at[sl, :], acc_v)
        pltpu.sync_copy(rc1_hbm.at[wid, ch], idx1_v)
        pltpu.sync_copy(rc2_hbm.at[wid, ch], idx2_v)
        cp1 = pltpu.async_copy(xo_hbm.at[idx1_v], g1_v, sem)
        cp2 = pltpu.async_copy(xo_hbm.at[idx2_v], g2_v, sem)
        cp1.wait()
        cp2.wait()

        def tok_body(i, _):
            def vec_body(j, _):
                off = j * 16
                a = acc_v[i, pl.ds(off, 16)]
                a = a + g1_v[i, pl.ds(off, 16)] + g2_v[i, pl.ds(off, 16)]
                acc_v[i, pl.ds(off, 16)] = a
                return 0

            return lax.fori_loop(0, _D // 16, vec_body, 0)

        lax.fori_loop(0, _CH, tok_body, 0)
        pltpu.sync_copy(acc_v, y_hbm.at[sl, :])


def _combine(xdout, shared_out, rc1, rc2):
    mesh = plsc.VectorSubcoreMesh(core_axis_name="c", subcore_axis_name="s")
    return pl.kernel(
        _combine_body,
        out_type=jax.ShapeDtypeStruct((_T, _D), jnp.float32),
        mesh=mesh,
        scratch_types=[
            pltpu.VMEM((_CH, _D), jnp.float32),
            pltpu.VMEM((_CH, _D), jnp.float32),
            pltpu.VMEM((_CH, _D), jnp.float32),
            pltpu.VMEM((_CH,), jnp.int32),
            pltpu.VMEM((_CH,), jnp.int32),
            pltpu.SemaphoreType.DMA,
        ],
    )(xdout, shared_out, rc1, rc2)


def kernel(x, shared_gate_w, shared_up_w, shared_down_w, gate_w, logit_bias,
           null_logit, W_gate, W_up, W_down):
    Bx, Tx, D = x.shape
    x2d = x.reshape(_T, _D)
    (rs1, rs2, rc1, rc2, w1row, w2row, be2d, nb2d, aux) = _gating(
        x2d, gate_w, logit_bias, null_logit)
    nchunk = _TPW // _CH
    rs1 = rs1.reshape(_NW, nchunk, _CH)
    rs2 = rs2.reshape(_NW, nchunk, _CH)
    rc1 = rc1.reshape(_NW, nchunk, _CH)
    rc2 = rc2.reshape(_NW, nchunk, _CH)
    be = be2d.reshape(_NB + 1)
    nb = nb2d.reshape(1)

    xd, wd_rows = _scatter(x2d, w1row, w2row, rs1, rs2)
    shared_out = _shared_ffn(x2d, shared_gate_w, shared_up_w, shared_down_w)
    xdout = _experts(be, nb, xd, wd_rows, W_gate, W_up, W_down)
    y = _combine(xdout, shared_out, rc1, rc2)
    return y.reshape(Bx, Tx, D), aux[0, 0]


# speedup vs baseline: 1.0595x; 1.0595x over previous
import functools

import jax
import jax.numpy as jnp
from jax import lax
from jax.experimental import pallas as pl
from jax.experimental.pallas import tpu as pltpu
from jax.experimental.pallas import tpu_sc as plsc

_T = 2048
_D = 1024
_DH = 512
_DS = 2048
_E = 8
_NULL_COPIES = 8
_RHO = 0.5

_BLK = 256
_NB = 24
_DR = _NB * _BLK
_DR_TOT = _DR + _BLK
_TRASH = _DR
_NW = 32
_TPW = _T // _NW
_CH = 32


def _silu(v):
    return v * jax.nn.sigmoid(v)


def _gating_body(x_ref, gw_ref, bias_ref, null_ref, rs1_ref, rs2_ref,
                 rc1_ref, rc2_ref, wc1_ref, wc2_ref, be_ref, nb_ref, aux_ref):
    x = x_ref[...]
    logits = jnp.dot(x, gw_ref[...], preferred_element_type=jnp.float32)
    logits = logits + bias_ref[...]
    n = null_ref[0, 0]
    T, E = logits.shape

    iota = lax.broadcasted_iota(jnp.int32, (T, E), 1)
    v1 = jnp.max(logits, axis=1, keepdims=True)
    e1 = jnp.min(jnp.where(logits == v1, iota, E), axis=1, keepdims=True)
    l2 = jnp.where(iota == e1, -jnp.inf, logits)
    v2 = jnp.max(l2, axis=1, keepdims=True)
    e2 = jnp.min(jnp.where(l2 == v2, iota, E), axis=1, keepdims=True)

    t1_real = v1 >= n
    t2_real = jnp.logical_and(t1_real, v2 >= n)

    w1_both = 1.0 / (1.0 + jnp.exp(v2 - v1))
    w1 = jnp.where(t1_real, jnp.where(t2_real, w1_both, 1.0), 0.0)
    w2 = jnp.where(t2_real, 1.0 - w1_both, 0.0)
    wc1_ref[...] = jnp.broadcast_to(w1, (T, 128))
    wc2_ref[...] = jnp.broadcast_to(w2, (T, 128))

    f32 = jnp.float32
    bf = jnp.bfloat16
    oh1 = jnp.where(jnp.logical_and(iota == e1, t1_real), 1.0, 0.0)
    oh2 = jnp.where(jnp.logical_and(iota == e2, t2_real), 1.0, 0.0)
    hcnt = oh1 + oh2

    r_io = lax.broadcasted_iota(jnp.int32, (T, T), 0)
    c_io = lax.broadcasted_iota(jnp.int32, (T, T), 1)
    lst = jnp.where(r_io > c_io, 1.0, 0.0).astype(bf)
    cpre = jnp.dot(lst, hcnt.astype(bf), preferred_element_type=f32)

    tot = jnp.sum(hcnt, axis=0, keepdims=True)
    padded = jnp.floor((tot + (_BLK - 1)) * (1.0 / _BLK)) * _BLK
    er = lax.broadcasted_iota(jnp.int32, (E, E), 0)
    ec = lax.broadcasted_iota(jnp.int32, (E, E), 1)
    su = jnp.where(er < ec, 1.0, 0.0).astype(bf)
    base = jnp.dot(padded.astype(bf), su, preferred_element_type=f32)

    pos1 = jnp.sum(oh1 * (cpre + base), axis=1, keepdims=True)
    pos2 = jnp.sum(oh2 * (cpre + base), axis=1, keepdims=True)
    p1i = pos1.astype(jnp.int32)
    p2i = pos2.astype(jnp.int32)
    rs1_ref[...] = jnp.where(t1_real, p1i, _TRASH)
    rs2_ref[...] = jnp.where(t2_real, p2i, _TRASH)
    rc1_ref[...] = jnp.where(t1_real, p1i, _DR)
    rc2_ref[...] = jnp.where(t2_real, p2i, _DR)

    ends = base + padded
    jv = (lax.broadcasted_iota(jnp.int32, (_NB + 1, E), 0) * _BLK).astype(f32)
    beq = jnp.sum(jnp.where(jnp.broadcast_to(ends, (_NB + 1, E)) <= jv, 1, 0),
                  axis=1, keepdims=True)
    ev = lax.broadcasted_iota(jnp.int32, (1, E), 1)
    last_e = jnp.max(jnp.where(padded > 0, ev, 0))
    be_ref[...] = jnp.minimum(beq, last_e).reshape(1, _NB + 1)
    nb_ref[0, 0] = (jnp.sum(padded) * (1.0 / _BLK)).astype(jnp.int32)

    p = jnp.exp(logits - v1)
    probs_real = p / jnp.sum(p, axis=1, keepdims=True)
    P_real = jnp.sum(probs_real, axis=0) / T

    counts = jnp.sum(hcnt, axis=0)
    total_real = jnp.maximum(jnp.sum(counts), 1e-6)
    f_real = counts / total_real
    L_bal = E * jnp.sum(f_real * P_real)

    null_slots = jnp.sum(jnp.where(t1_real, 0.0, 1.0) + jnp.where(t2_real, 0.0, 1.0))
    null_rate = null_slots / (T * 2)
    L_null = (null_rate - _RHO) ** 2

    m = jnp.maximum(v1, n)
    s_all = (jnp.sum(jnp.exp(logits - m), axis=1, keepdims=True)
             + _NULL_COPIES * jnp.exp(n - m))
    lse = m + jnp.log(s_all)
    L_z = jnp.sum(lse * lse) / T

    aux_ref[0, 0] = 0.02 * L_bal + 0.001 * L_z + 0.01 * L_null


def _gating(x2d, gate_w, logit_bias, null_logit):
    i32 = jnp.int32
    f32 = jnp.float32
    return pl.pallas_call(
        _gating_body,
        out_shape=(
            jax.ShapeDtypeStruct((_T, 1), i32),
            jax.ShapeDtypeStruct((_T, 1), i32),
            jax.ShapeDtypeStruct((_T, 1), i32),
            jax.ShapeDtypeStruct((_T, 1), i32),
            jax.ShapeDtypeStruct((_T, 128), f32),
            jax.ShapeDtypeStruct((_T, 128), f32),
            jax.ShapeDtypeStruct((1, _NB + 1), i32),
            jax.ShapeDtypeStruct((1, 1), i32),
            jax.ShapeDtypeStruct((1, 1), f32),
        ),
        in_specs=[pl.BlockSpec(memory_space=pltpu.VMEM)] * 4,
        out_specs=(
            pl.BlockSpec(memory_space=pltpu.VMEM),
            pl.BlockSpec(memory_space=pltpu.VMEM),
            pl.BlockSpec(memory_space=pltpu.VMEM),
            pl.BlockSpec(memory_space=pltpu.VMEM),
            pl.BlockSpec(memory_space=pltpu.VMEM),
            pl.BlockSpec(memory_space=pltpu.VMEM),
            pl.BlockSpec(memory_space=pltpu.VMEM),
            pl.BlockSpec(memory_space=pltpu.SMEM),
            pl.BlockSpec(memory_space=pltpu.SMEM),
        ),
    )(x2d, gate_w, logit_bias.reshape(1, _E), null_logit.reshape(1, 1))


def _shared_body(x_ref, gw_ref, uw_ref, dw_ref, out_ref):
    x = x_ref[...].astype(jnp.bfloat16)
    g = jnp.dot(x, gw_ref[...].astype(jnp.bfloat16), preferred_element_type=jnp.float32)
    u = jnp.dot(x, uw_ref[...].astype(jnp.bfloat16), preferred_element_type=jnp.float32)
    h = (_silu(g) * u).astype(jnp.bfloat16)
    out_ref[...] = jnp.dot(h, dw_ref[...].astype(jnp.bfloat16),
                           preferred_element_type=jnp.float32)


def _shared_ffn(x2d, gw, uw, dw, tb=512):
    grid = (_T // tb,)
    return pl.pallas_call(
        _shared_body,
        grid=grid,
        out_shape=jax.ShapeDtypeStruct((_T, _D), jnp.float32),
        in_specs=[
            pl.BlockSpec((tb, _D), lambda t: (t, 0)),
            pl.BlockSpec((_D, _DS), lambda t: (0, 0)),
            pl.BlockSpec((_D, _DS), lambda t: (0, 0)),
            pl.BlockSpec((_DS, _D), lambda t: (0, 0)),
        ],
        out_specs=pl.BlockSpec((tb, _D), lambda t: (t, 0)),
    )(x2d, gw, uw, dw)


def _sc_wid():
    return lax.axis_index("s") * 2 + lax.axis_index("c")


def _scatter_body(x_hbm, w1_hbm, w2_hbm, rs1_hbm, rs2_hbm, xd_hbm, wd_hbm,
                  rows_v, wr1_v, wr2_v, idx1_v, idx2_v, sem):
    wid = _sc_wid()
    base = wid * _TPW
    pltpu.sync_copy(rs1_hbm.at[wid], idx1_v)
    pltpu.sync_copy(rs2_hbm.at[wid], idx2_v)
    for ch in range(_TPW // _CH):
        sl = pl.ds(base + ch * _CH, _CH)
        pltpu.sync_copy(x_hbm.at[sl, :], rows_v)
        pltpu.sync_copy(w1_hbm.at[sl, :], wr1_v)
        pltpu.sync_copy(w2_hbm.at[sl, :], wr2_v)
        cp1 = pltpu.async_copy(rows_v, xd_hbm.at[idx1_v.at[ch]], sem)
        cp2 = pltpu.async_copy(rows_v, xd_hbm.at[idx2_v.at[ch]], sem)
        cp3 = pltpu.async_copy(wr1_v, wd_hbm.at[idx1_v.at[ch]], sem)
        cp4 = pltpu.async_copy(wr2_v, wd_hbm.at[idx2_v.at[ch]], sem)
        cp1.wait()
        cp2.wait()
        cp3.wait()
        cp4.wait()


def _scatter(x2d, w1row, w2row, rs1, rs2):
    mesh = plsc.VectorSubcoreMesh(core_axis_name="c", subcore_axis_name="s")
    return pl.kernel(
        _scatter_body,
        out_type=(
            jax.ShapeDtypeStruct((_DR_TOT, _D), jnp.float32),
            jax.ShapeDtypeStruct((_DR_TOT, 128), jnp.float32),
        ),
        mesh=mesh,
        scratch_types=[
            pltpu.VMEM((_CH, _D), jnp.float32),
            pltpu.VMEM((_CH, 128), jnp.float32),
            pltpu.VMEM((_CH, 128), jnp.float32),
            pltpu.VMEM((_TPW // _CH, _CH), jnp.int32),
            pltpu.VMEM((_TPW // _CH, _CH), jnp.int32),
            pltpu.SemaphoreType.DMA,
        ],
    )(x2d, w1row, w2row, rs1, rs2)


def _experts_body(be_ref, nb_ref, xd_ref, wrow_ref, wg_ref, wu_ref, wd_ref,
                  out_ref):
    b = pl.program_id(0)

    @pl.when(b < nb_ref[0])
    def _():
        xb = xd_ref[...].astype(jnp.bfloat16)
        g = jnp.dot(xb, wg_ref[0].astype(jnp.bfloat16),
                    preferred_element_type=jnp.float32)
        u = jnp.dot(xb, wu_ref[0].astype(jnp.bfloat16),
                    preferred_element_type=jnp.float32)
        w = wrow_ref[:, 0:1]
        h = (_silu(g) * u * w).astype(jnp.bfloat16)
        out_ref[...] = jnp.dot(h, wd_ref[0].astype(jnp.bfloat16),
                               preferred_element_type=jnp.float32)

    @pl.when(b == _NB)
    def _zero_tail():
        out_ref[...] = jnp.zeros((_BLK, _D), jnp.float32)


def _experts(be, nb, xd, wd_rows, W_gate, W_up, W_down):
    grid_spec = pltpu.PrefetchScalarGridSpec(
        num_scalar_prefetch=2,
        grid=(_NB + 1,),
        in_specs=[
            pl.BlockSpec((_BLK, _D), lambda b, be, nb: (b, 0)),
            pl.BlockSpec((_BLK, 128), lambda b, be, nb: (b, 0)),
            pl.BlockSpec((1, _D, _DH), lambda b, be, nb: (be[b], 0, 0)),
            pl.BlockSpec((1, _D, _DH), lambda b, be, nb: (be[b], 0, 0)),
            pl.BlockSpec((1, _DH, _D), lambda b, be, nb: (be[b], 0, 0)),
        ],
        out_specs=pl.BlockSpec((_BLK, _D), lambda b, be, nb: (b, 0)),
    )
    return pl.pallas_call(
        _experts_body,
        grid_spec=grid_spec,
        out_shape=jax.ShapeDtypeStruct((_DR_TOT, _D), jnp.float32),
    )(be, nb, xd, wd_rows, W_gate, W_up, W_down)


def _combine_body(xo_hbm, sh_hbm, rc1_hbm, rc2_hbm, y_hbm,
                  acc_v, g1_v, g2_v, idx1_v, idx2_v, sem):
    wid = _sc_wid()
    base = wid * _TPW
    for ch in range(_TPW // _CH):
        sl = pl.ds(base + ch * _CH, _CH)
        pltpu.sync_copy(sh_hbm.at[sl, :], acc_v)
        pltpu.sync_copy(rc1_hbm.at[wid, ch], idx1_v)
        pltpu.sync_copy(rc2_hbm.at[wid, ch], idx2_v)
        cp1 = pltpu.async_copy(xo_hbm.at[idx1_v], g1_v, sem)
        cp2 = pltpu.async_copy(xo_hbm.at[idx2_v], g2_v, sem)
        cp1.wait()
        cp2.wait()

        def tok_body(i, _):
            for j in range(_D // 16):
                off = j * 16
                a = acc_v[i, pl.ds(off, 16)]
                a = a + g1_v[i, pl.ds(off, 16)] + g2_v[i, pl.ds(off, 16)]
                acc_v[i, pl.ds(off, 16)] = a
            return 0

        lax.fori_loop(0, _CH, tok_body, 0)
        pltpu.sync_copy(acc_v, y_hbm.at[sl, :])


def _combine(xdout, shared_out, rc1, rc2):
    mesh = plsc.VectorSubcoreMesh(core_axis_name="c", subcore_axis_name="s")
    return pl.kernel(
        _combine_body,
        out_type=jax.ShapeDtypeStruct((_T, _D), jnp.float32),
        mesh=mesh,
        scratch_types=[
            pltpu.VMEM((_CH, _D), jnp.float32),
            pltpu.VMEM((_CH, _D), jnp.float32),
            pltpu.VMEM((_CH, _D), jnp.float32),
            pltpu.VMEM((_CH,), jnp.int32),
            pltpu.VMEM((_CH,), jnp.int32),
            pltpu.SemaphoreType.DMA,
        ],
    )(xdout, shared_out, rc1, rc2)


def kernel(x, shared_gate_w, shared_up_w, shared_down_w, gate_w, logit_bias,
           null_logit, W_gate, W_up, W_down):
    Bx, Tx, D = x.shape
    x2d = x.reshape(_T, _D)
    (rs1, rs2, rc1, rc2, w1row, w2row, be2d, nb2d, aux) = _gating(
        x2d, gate_w, logit_bias, null_logit)
    nchunk = _TPW // _CH
    rs1 = rs1.reshape(_NW, nchunk, _CH)
    rs2 = rs2.reshape(_NW, nchunk, _CH)
    rc1 = rc1.reshape(_NW, nchunk, _CH)
    rc2 = rc2.reshape(_NW, nchunk, _CH)
    be = be2d.reshape(_NB + 1)
    nb = nb2d.reshape(1)

    xd, wd_rows = _scatter(x2d, w1row, w2row, rs1, rs2)
    shared_out = _shared_ffn(x2d, shared_gate_w, shared_up_w, shared_down_w)
    xdout = _experts(be, nb, xd, wd_rows, W_gate, W_up, W_down)
    y = _combine(xdout, shared_out, rc1, rc2)
    return y.reshape(Bx, Tx, D), aux[0, 0]
